# baseline (speedup 1.0000x reference)
import functools

import jax
import jax.numpy as jnp
from jax import lax
from jax.experimental import pallas as pl
from jax.experimental.pallas import tpu as pltpu
from jax.experimental.pallas import tpu_sc as plsc

N_ATOMS = 10000
N_BONDS = 160000
MAX_NB = 16
ATOM_FDIM = 256
BOND_FDIM = 272
HIDDEN = 256
DEPTH = 4
N_MOLS = 500

NC = 2
NS = 16
NW = NC * NS

A_PAD = 10240
B_PAD = 163840
M_PAD = 512

A_PER_W = A_PAD // NW
CH_A = 8
N_CH_A = A_PER_W // CH_A
NB_A = 4

B_PER_W = B_PAD // NW
CH_B = 40
N_CH_B = B_PER_W // CH_B
NB_B = 4

LG = HIDDEN // 16


def _rne_hi16(u):
    return (u + jnp.uint32(0x7FFF) + ((u >> 16) & jnp.uint32(1))) \
        & jnp.uint32(0xFFFF0000)


def _pack_tc(y):
    ua = jax.lax.bitcast_convert_type(y[:, :128], jnp.uint32)
    ub = jax.lax.bitcast_convert_type(y[:, 128:], jnp.uint32)
    return (_rne_hi16(ua) >> 16) | _rne_hi16(ub)


def _unpack_tc(u, relu):
    lo = jax.lax.bitcast_convert_type(u << 16, jnp.float32)
    hi = jax.lax.bitcast_convert_type(u & jnp.uint32(0xFFFF0000), jnp.float32)
    if relu:
        lo = jnp.maximum(lo, 0.0)
        hi = jnp.maximum(hi, 0.0)
    return lo.astype(jnp.bfloat16), hi.astype(jnp.bfloat16)


def _mm_body(x_ref, w_ref, o_ref, *, relu_in, packed_in):
    w = w_ref[...].astype(jnp.bfloat16)
    if packed_in:
        lo, hi = _unpack_tc(x_ref[...], relu_in)
        y = (jnp.dot(lo, w[:128], preferred_element_type=jnp.float32)
             + jnp.dot(hi, w[128:], preferred_element_type=jnp.float32))
    else:
        x = x_ref[...]
        if relu_in:
            x = jnp.maximum(x, 0.0)
        y = jnp.dot(x.astype(jnp.bfloat16), w,
                    preferred_element_type=jnp.float32)
    o_ref[...] = _pack_tc(y)


def _mm(x, w, relu_in=False, bm=512, out_m=None):
    m, k = x.shape
    n = w.shape[1]
    packed_in = x.dtype == jnp.uint32
    out_m = m if out_m is None else out_m
    return pl.pallas_call(
        functools.partial(_mm_body, relu_in=relu_in, packed_in=packed_in),
        grid=(m // bm,),
        in_specs=[pl.BlockSpec((bm, k), lambda i: (i, 0)),
                  pl.BlockSpec(w.shape, lambda i: (0, 0))],
        out_specs=pl.BlockSpec((bm, n // 2), lambda i: (i, 0)),
        out_shape=jax.ShapeDtypeStruct((out_m, n // 2), jnp.uint32),
    )(x, w)


def _readout_body(fa_ref, am_ref, ids_ref, wo_ref, bo_ref, o_ref,
                  sums_ref, cnts_ref):
    step = pl.program_id(0)

    @pl.when(step == 0)
    def _init():
        sums_ref[...] = jnp.zeros_like(sums_ref)
        cnts_ref[...] = jnp.zeros_like(cnts_ref)

    wo = wo_ref[...].astype(jnp.bfloat16)
    lo, hi = _unpack_tc(am_ref[...], False)
    hid = (jnp.dot(fa_ref[...].astype(jnp.bfloat16), wo[:ATOM_FDIM],
                   preferred_element_type=jnp.float32)
           + jnp.dot(lo, wo[ATOM_FDIM:ATOM_FDIM + 128],
                     preferred_element_type=jnp.float32)
           + jnp.dot(hi, wo[ATOM_FDIM + 128:],
                     preferred_element_type=jnp.float32)
           + bo_ref[...])
    hid = jnp.maximum(hid, 0.0)

    ids = ids_ref[...]
    mol_iota = lax.broadcasted_iota(jnp.int32, (M_PAD, ids.shape[1]), 0)
    onehot_t = (mol_iota == ids).astype(jnp.float32)
    sums_ref[...] += jnp.dot(onehot_t, hid, preferred_element_type=jnp.float32)
    cnts_ref[...] += jnp.broadcast_to(
        jnp.sum(onehot_t, axis=1, keepdims=True), cnts_ref.shape)

    @pl.when(step == pl.num_programs(0) - 1)
    def _fin():
        o_ref[...] = sums_ref[...] / jnp.maximum(cnts_ref[:, 0:1], 1.0)


def _readout(f_atoms_p, am_f, ids_p, w_o, b_o, bm=512):
    grid = (A_PAD // bm,)
    return pl.pallas_call(
        _readout_body,
        grid=grid,
        in_specs=[pl.BlockSpec((bm, ATOM_FDIM), lambda i: (i, 0)),
                  pl.BlockSpec((bm, HIDDEN // 2), lambda i: (i, 0)),
                  pl.BlockSpec((1, bm), lambda i: (0, i)),
                  pl.BlockSpec((ATOM_FDIM + HIDDEN, HIDDEN), lambda i: (0, 0)),
                  pl.BlockSpec((1, HIDDEN), lambda i: (0, 0))],
        out_specs=pl.BlockSpec((M_PAD, HIDDEN), lambda i: (0, 0)),
        out_shape=jax.ShapeDtypeStruct((M_PAD, HIDDEN), jnp.float32),
        scratch_shapes=[pltpu.VMEM((M_PAD, HIDDEN), jnp.float32),
                        pltpu.VMEM((M_PAD, 128), jnp.float32)],
    )(f_atoms_p, am_f, ids_p, w_o, b_o.reshape(1, HIDDEN))


PK = HIDDEN // 2
PKG = PK // 16


def _bf16_split(u):
    lo = plsc.bitcast(u << 16, jnp.float32)
    hi = plsc.bitcast(u & jnp.uint32(0xFFFF0000), jnp.float32)
    return lo, hi


def _bf16_join(lo, hi):
    ul = plsc.bitcast(lo, jnp.uint32)
    uh = plsc.bitcast(hi, jnp.uint32)
    return (_rne_hi16(ul) >> 16) | _rne_hi16(uh)


def _bf16_join_trunc(lo, hi):
    ul = plsc.bitcast(lo, jnp.uint32)
    uh = plsc.bitcast(hi, jnp.uint32)
    return (ul >> 16) | (uh & jnp.uint32(0xFFFF0000))

def _gathersum_sc(tab, a2b_flat, relu_in):
    mesh = plsc.VectorSubcoreMesh(core_axis_name="c", subcore_axis_name="s")

    @functools.partial(
        pl.kernel, mesh=mesh,
        compiler_params=pltpu.CompilerParams(needs_layout_passes=False),
        out_type=jax.ShapeDtypeStruct((A_PAD, PK), jnp.uint32),
        scratch_types=[
            pltpu.VMEM((A_PER_W * MAX_NB,), jnp.int32),
            pltpu.VMEM((CH_A * MAX_NB, PK), jnp.uint32),
            pltpu.VMEM((CH_A * MAX_NB, PK), jnp.uint32),
            pltpu.VMEM((CH_A * MAX_NB, PK), jnp.uint32),
            pltpu.VMEM((CH_A * MAX_NB, PK), jnp.uint32),
            pltpu.VMEM((A_PER_W, PK), jnp.uint32),
            pltpu.SemaphoreType.DMA,
            pltpu.SemaphoreType.DMA,
            pltpu.SemaphoreType.DMA,
            pltpu.SemaphoreType.DMA,
        ],
    )
    def k(tab_hbm, idx_hbm, out_hbm, idx_all, rows0, rows1, rows2, rows3,
          out_all, gsem0, gsem1, gsem2, gsem3):
        wid = lax.axis_index("s") * NC + lax.axis_index("c")
        rows = (rows0, rows1, rows2, rows3)
        gsems = (gsem0, gsem1, gsem2, gsem3)

        pltpu.sync_copy(idx_hbm.at[pl.ds(wid * A_PER_W * MAX_NB,
                                         A_PER_W * MAX_NB)], idx_all)

        def issue(g, b):
            iv = idx_all.at[pl.ds(g * (CH_A * MAX_NB), CH_A * MAX_NB)]
            pltpu.async_copy(tab_hbm.at[iv], rows[b], gsems[b])

        def drain(b):
            pltpu.make_async_copy(tab_hbm.at[pl.ds(0, CH_A * MAX_NB)],
                                  rows[b], gsems[b]).wait()

        for b in range(NB_A):
            issue(b, b)

        def step(g2, carry):
            for b in range(NB_A):
                g = NB_A * g2 + b
                drain(b)

                def red(a, c2):
                    def ld(r, c):
                        lo, hi = _bf16_split(
                            rows[b][a * MAX_NB + r, pl.ds(c * 16, 16)])
                        if relu_in:
                            lo = jnp.maximum(lo, 0.0)
                            hi = jnp.maximum(hi, 0.0)
                        return lo, hi

                    for c0 in range(0, PKG, 4):
                        cg = range(c0, c0 + 4)
                        accs = {c: ld(0, c) for c in cg}
                        for r in range(1, MAX_NB):
                            for c in cg:
                                e, o = ld(r, c)
                                accs[c] = (accs[c][0] + e, accs[c][1] + o)
                        for c in cg:
                            out_all[g * CH_A + a, pl.ds(c * 16, 16)] = (
                                _bf16_join(accs[c][0], accs[c][1]))
                    return c2

                lax.fori_loop(0, CH_A, red, 0)

                @pl.when(g + NB_A < N_CH_A)
                def _():
                    issue(g + NB_A, b)
            return carry

        lax.fori_loop(0, N_CH_A // NB_A, step, 0)
        pltpu.sync_copy(out_all, out_hbm.at[pl.ds(wid * A_PER_W, A_PER_W)])

    return k(tab, a2b_flat)


_ST_BYTES = CH_B * (HIDDEN // 2) * 4


def _update_sc(inp, a_tab, mh_tab, b2a_p, b2revb_p):
    mesh = plsc.VectorSubcoreMesh(core_axis_name="c", subcore_axis_name="s")

    buf = lambda: [pltpu.VMEM((CH_B, PK), jnp.uint32) for _ in range(NB_B)]
    sem = lambda: [pltpu.SemaphoreType.DMA for _ in range(NB_B)]

    @functools.partial(
        pl.kernel, mesh=mesh,
        compiler_params=pltpu.CompilerParams(needs_layout_passes=False),
        out_type=jax.ShapeDtypeStruct((B_PAD, PK), jnp.uint32),
        scratch_types=[
            pltpu.VMEM((B_PER_W,), jnp.int32),
            pltpu.VMEM((B_PER_W,), jnp.int32),
            buf(), buf(), buf(), buf(),
            sem(), sem(), sem(), sem(),
        ],
    )
    def k(inp_hbm, a_hbm, mh_hbm, b2a_hbm, b2revb_hbm, out_hbm,
          idxa_all, idxr_all, ras, rrs, ris, ros, sas, srs, sis, sos):
        wid = lax.axis_index("s") * NC + lax.axis_index("c")
        wbase = wid * B_PER_W

        pltpu.sync_copy(b2a_hbm.at[pl.ds(wbase, B_PER_W)], idxa_all)
        pltpu.sync_copy(b2revb_hbm.at[pl.ds(wbase, B_PER_W)], idxr_all)

        def issue(g, b):
            iva = idxa_all.at[pl.ds(g * CH_B, CH_B)]
            ivr = idxr_all.at[pl.ds(g * CH_B, CH_B)]
            pltpu.async_copy(a_hbm.at[iva], ras[b], sas[b])
            pltpu.async_copy(mh_hbm.at[ivr], rrs[b], srs[b])
            pltpu.async_copy(inp_hbm.at[pl.ds(wbase + g * CH_B, CH_B)],
                             ris[b], sis[b])

        def drain_in(b):
            pltpu.make_async_copy(a_hbm.at[pl.ds(0, CH_B)], ras[b],
                                  sas[b]).wait()
            pltpu.make_async_copy(mh_hbm.at[pl.ds(0, CH_B)], rrs[b],
                                  srs[b]).wait()
            pltpu.make_async_copy(inp_hbm.at[pl.ds(0, CH_B)], ris[b],
                                  sis[b]).wait()

        def drain_out(b):
            pltpu.make_async_copy(ros[b], out_hbm.at[pl.ds(0, CH_B)],
                                  sos[b]).wait()

        for b in range(NB_B):
            issue(b, b)

        def step(g2, carry):
            for b in range(NB_B):
                g = NB_B * g2 + b
                drain_in(b)

                @pl.when(g >= NB_B)
                def _():
                    drain_out(b)

                def ew(r, c2):
                    for c in range(PKG):
                        s = pl.ds(c * 16, 16)
                        il, ih = _bf16_split(ris[b][r, s])
                        al, ah = _bf16_split(ras[b][r, s])
                        ml, mh_ = _bf16_split(rrs[b][r, s])
                        vl = jnp.maximum(il + al - ml, 0.0)
                        vh = jnp.maximum(ih + ah - mh_, 0.0)
                        ros[b][r, s] = _bf16_join(vl, vh)
                    return c2

                lax.fori_loop(0, CH_B, ew, 0)
                pltpu.async_copy(ros[b],
                                 out_hbm.at[pl.ds(wbase + g * CH_B, CH_B)],
                                 sos[b])

                @pl.when(g + NB_B < N_CH_B)
                def _():
                    issue(g + NB_B, b)
            return carry

        lax.fori_loop(0, N_CH_B // NB_B, step, 0)
        for b in range(NB_B):
            drain_out(b)

    return k(inp, a_tab, mh_tab, b2a_p, b2revb_p)


def kernel(f_atoms, f_bonds, a2b, b2a, b2revb, mol_segment_ids,
           W_i, W_h, W_o, b_o):
    f_atoms_p = jnp.pad(f_atoms, ((0, A_PAD - N_ATOMS), (0, 0)))

    nbp = B_PAD - N_BONDS
    nap = A_PAD - N_ATOMS
    b2a_p = jnp.concatenate(
        [b2a.astype(jnp.int32),
         (jnp.arange(nbp, dtype=jnp.int32) * 131) % N_ATOMS])
    b2revb_p = jnp.concatenate(
        [b2revb.astype(jnp.int32),
         (jnp.arange(nbp, dtype=jnp.int32) * 97) % N_BONDS])
    a2b_p = jnp.concatenate(
        [a2b.astype(jnp.int32).reshape(-1),
         (jnp.arange(nap * MAX_NB, dtype=jnp.int32) * 193) % N_BONDS])
    ids_p = jnp.concatenate(
        [mol_segment_ids.astype(jnp.int32),
         jnp.full((nap,), M_PAD - 1, jnp.int32)]).reshape(1, A_PAD)

    inp = _mm(f_bonds, W_i, bm=3200, out_m=B_PAD)

    msg = inp
    relu_first = True
    for _ in range(DEPTH - 1):
        mh = _mm(msg, W_h, relu_in=relu_first, bm=4096)
        am = _gathersum_sc(msg, a2b_p, relu_in=relu_first)
        a_tab = _mm(am, W_h, bm=2048)
        msg = _update_sc(inp, a_tab, mh, b2a_p, b2revb_p)
        relu_first = False

    am_f = _gathersum_sc(msg, a2b_p, relu_in=False)
    mol = _readout(f_atoms_p, am_f, ids_p, W_o, b_o)
    return mol[:N_MOLS]


# speedup vs baseline: 1.1238x; 1.1238x over previous
import functools

import jax
import jax.numpy as jnp
from jax import lax
from jax.experimental import pallas as pl
from jax.experimental.pallas import tpu as pltpu
from jax.experimental.pallas import tpu_sc as plsc

N_ATOMS = 10000
N_BONDS = 160000
MAX_NB = 16
ATOM_FDIM = 256
BOND_FDIM = 272
HIDDEN = 256
DEPTH = 4
N_MOLS = 500

NC = 2
NS = 16
NW = NC * NS

A_PAD = 10240
B_PAD = 163840
M_PAD = 512

A_PER_W = A_PAD // NW
CH_A = 8
N_CH_A = A_PER_W // CH_A
NB_A = 4

B_PER_W = B_PAD // NW
CH_B = 40
N_CH_B = B_PER_W // CH_B
NB_B = 4

LG = HIDDEN // 16


def _rne_hi16(u):
    return (u + jnp.uint32(0x7FFF) + ((u >> 16) & jnp.uint32(1))) \
        & jnp.uint32(0xFFFF0000)


def _pack_tc(y):
    ua = jax.lax.bitcast_convert_type(y[:, :128], jnp.uint32)
    ub = jax.lax.bitcast_convert_type(y[:, 128:], jnp.uint32)
    return (_rne_hi16(ua) >> 16) | _rne_hi16(ub)


def _unpack_tc(u, relu):
    lo = jax.lax.bitcast_convert_type(u << 16, jnp.float32)
    hi = jax.lax.bitcast_convert_type(u & jnp.uint32(0xFFFF0000), jnp.float32)
    if relu:
        lo = jnp.maximum(lo, 0.0)
        hi = jnp.maximum(hi, 0.0)
    return lo.astype(jnp.bfloat16), hi.astype(jnp.bfloat16)


def _mm_body(x_ref, w_ref, o_ref, *, relu_in, packed_in):
    w = w_ref[...].astype(jnp.bfloat16)
    if packed_in:
        lo, hi = _unpack_tc(x_ref[...], relu_in)
        y = (jnp.dot(lo, w[:128], preferred_element_type=jnp.float32)
             + jnp.dot(hi, w[128:], preferred_element_type=jnp.float32))
    else:
        x = x_ref[...]
        if relu_in:
            x = jnp.maximum(x, 0.0)
        y = jnp.dot(x.astype(jnp.bfloat16), w,
                    preferred_element_type=jnp.float32)
    o_ref[...] = _pack_tc(y)


def _mm(x, w, relu_in=False, bm=512, out_m=None):
    m, k = x.shape
    n = w.shape[1]
    packed_in = x.dtype == jnp.uint32
    out_m = m if out_m is None else out_m
    return pl.pallas_call(
        functools.partial(_mm_body, relu_in=relu_in, packed_in=packed_in),
        grid=(m // bm,),
        in_specs=[pl.BlockSpec((bm, k), lambda i: (i, 0)),
                  pl.BlockSpec(w.shape, lambda i: (0, 0))],
        out_specs=pl.BlockSpec((bm, n // 2), lambda i: (i, 0)),
        out_shape=jax.ShapeDtypeStruct((out_m, n // 2), jnp.uint32),
    )(x, w)


def _readout_body(fa_ref, am_ref, ids_ref, wo_ref, bo_ref, o_ref,
                  sums_ref, cnts_ref):
    step = pl.program_id(0)

    @pl.when(step == 0)
    def _init():
        sums_ref[...] = jnp.zeros_like(sums_ref)
        cnts_ref[...] = jnp.zeros_like(cnts_ref)

    wo = wo_ref[...].astype(jnp.bfloat16)
    lo, hi = _unpack_tc(am_ref[...], False)
    hid = (jnp.dot(fa_ref[...].astype(jnp.bfloat16), wo[:ATOM_FDIM],
                   preferred_element_type=jnp.float32)
           + jnp.dot(lo, wo[ATOM_FDIM:ATOM_FDIM + 128],
                     preferred_element_type=jnp.float32)
           + jnp.dot(hi, wo[ATOM_FDIM + 128:],
                     preferred_element_type=jnp.float32)
           + bo_ref[...])
    hid = jnp.maximum(hid, 0.0)

    ids = ids_ref[...]
    mol_iota = lax.broadcasted_iota(jnp.int32, (M_PAD, ids.shape[1]), 0)
    onehot_t = (mol_iota == ids).astype(jnp.float32)
    sums_ref[...] += jnp.dot(onehot_t, hid, preferred_element_type=jnp.float32)
    cnts_ref[...] += jnp.broadcast_to(
        jnp.sum(onehot_t, axis=1, keepdims=True), cnts_ref.shape)

    @pl.when(step == pl.num_programs(0) - 1)
    def _fin():
        o_ref[...] = sums_ref[...] / jnp.maximum(cnts_ref[:, 0:1], 1.0)


def _readout(f_atoms_p, am_f, ids_p, w_o, b_o, bm=512):
    grid = (A_PAD // bm,)
    return pl.pallas_call(
        _readout_body,
        grid=grid,
        in_specs=[pl.BlockSpec((bm, ATOM_FDIM), lambda i: (i, 0)),
                  pl.BlockSpec((bm, HIDDEN // 2), lambda i: (i, 0)),
                  pl.BlockSpec((1, bm), lambda i: (0, i)),
                  pl.BlockSpec((ATOM_FDIM + HIDDEN, HIDDEN), lambda i: (0, 0)),
                  pl.BlockSpec((1, HIDDEN), lambda i: (0, 0))],
        out_specs=pl.BlockSpec((M_PAD, HIDDEN), lambda i: (0, 0)),
        out_shape=jax.ShapeDtypeStruct((M_PAD, HIDDEN), jnp.float32),
        scratch_shapes=[pltpu.VMEM((M_PAD, HIDDEN), jnp.float32),
                        pltpu.VMEM((M_PAD, 128), jnp.float32)],
    )(f_atoms_p, am_f, ids_p, w_o, b_o.reshape(1, HIDDEN))


PK = HIDDEN // 2
PKG = PK // 16


def _bf16_split(u):
    lo = plsc.bitcast(u << 16, jnp.float32)
    hi = plsc.bitcast(u & jnp.uint32(0xFFFF0000), jnp.float32)
    return lo, hi


def _bf16_join(lo, hi):
    ul = plsc.bitcast(lo, jnp.uint32)
    uh = plsc.bitcast(hi, jnp.uint32)
    return (_rne_hi16(ul) >> 16) | _rne_hi16(uh)


def _bf16_join_rta(lo, hi):
    ul = plsc.bitcast(lo, jnp.uint32) + jnp.uint32(0x8000)
    uh = plsc.bitcast(hi, jnp.uint32) + jnp.uint32(0x8000)
    return (ul >> 16) | (uh & jnp.uint32(0xFFFF0000))

def _gathersum_sc(tab, a2b_flat, relu_in):
    mesh = plsc.VectorSubcoreMesh(core_axis_name="c", subcore_axis_name="s")

    @functools.partial(
        pl.kernel, mesh=mesh,
        compiler_params=pltpu.CompilerParams(needs_layout_passes=False),
        out_type=jax.ShapeDtypeStruct((A_PAD, PK), jnp.uint32),
        scratch_types=[
            pltpu.VMEM((A_PER_W * MAX_NB,), jnp.int32),
            pltpu.VMEM((CH_A * MAX_NB, PK), jnp.uint32),
            pltpu.VMEM((CH_A * MAX_NB, PK), jnp.uint32),
            pltpu.VMEM((CH_A * MAX_NB, PK), jnp.uint32),
            pltpu.VMEM((CH_A * MAX_NB, PK), jnp.uint32),
            pltpu.VMEM((A_PER_W, PK), jnp.uint32),
            pltpu.SemaphoreType.DMA,
            pltpu.SemaphoreType.DMA,
            pltpu.SemaphoreType.DMA,
            pltpu.SemaphoreType.DMA,
        ],
    )
    def k(tab_hbm, idx_hbm, out_hbm, idx_all, rows0, rows1, rows2, rows3,
          out_all, gsem0, gsem1, gsem2, gsem3):
        wid = lax.axis_index("s") * NC + lax.axis_index("c")
        rows = (rows0, rows1, rows2, rows3)
        gsems = (gsem0, gsem1, gsem2, gsem3)

        pltpu.sync_copy(idx_hbm.at[pl.ds(wid * A_PER_W * MAX_NB,
                                         A_PER_W * MAX_NB)], idx_all)

        def issue(g, b):
            iv = idx_all.at[pl.ds(g * (CH_A * MAX_NB), CH_A * MAX_NB)]
            pltpu.async_copy(tab_hbm.at[iv], rows[b], gsems[b])

        def drain(b):
            pltpu.make_async_copy(tab_hbm.at[pl.ds(0, CH_A * MAX_NB)],
                                  rows[b], gsems[b]).wait()

        for b in range(NB_A):
            issue(b, b)

        def step(g2, carry):
            for b in range(NB_A):
                g = NB_A * g2 + b
                drain(b)

                def red(a, c2):
                    def ld(r, c):
                        lo, hi = _bf16_split(
                            rows[b][a * MAX_NB + r, pl.ds(c * 16, 16)])
                        if relu_in:
                            lo = jnp.maximum(lo, 0.0)
                            hi = jnp.maximum(hi, 0.0)
                        return lo, hi

                    for c0 in range(0, PKG, 4):
                        cg = range(c0, c0 + 4)
                        accs = {c: ld(0, c) for c in cg}
                        for r in range(1, MAX_NB):
                            for c in cg:
                                e, o = ld(r, c)
                                accs[c] = (accs[c][0] + e, accs[c][1] + o)
                        for c in cg:
                            out_all[g * CH_A + a, pl.ds(c * 16, 16)] = (
                                _bf16_join(accs[c][0], accs[c][1]))
                    return c2

                lax.fori_loop(0, CH_A, red, 0)

                @pl.when(g + NB_A < N_CH_A)
                def _():
                    issue(g + NB_A, b)
            return carry

        lax.fori_loop(0, N_CH_A // NB_A, step, 0)
        pltpu.sync_copy(out_all, out_hbm.at[pl.ds(wid * A_PER_W, A_PER_W)])

    return k(tab, a2b_flat)


_ST_BYTES = CH_B * (HIDDEN // 2) * 4


def _update_sc(inp, a_tab, mh_tab, b2a_p, b2revb_p):
    mesh = plsc.VectorSubcoreMesh(core_axis_name="c", subcore_axis_name="s")

    buf = lambda: [pltpu.VMEM((CH_B, PK), jnp.uint32) for _ in range(NB_B)]
    sem = lambda: [pltpu.SemaphoreType.DMA for _ in range(NB_B)]

    @functools.partial(
        pl.kernel, mesh=mesh,
        compiler_params=pltpu.CompilerParams(needs_layout_passes=False),
        out_type=jax.ShapeDtypeStruct((B_PAD, PK), jnp.uint32),
        scratch_types=[
            pltpu.VMEM((B_PER_W,), jnp.int32),
            pltpu.VMEM((B_PER_W,), jnp.int32),
            buf(), buf(), buf(), buf(),
            sem(), sem(), sem(), sem(),
        ],
    )
    def k(inp_hbm, a_hbm, mh_hbm, b2a_hbm, b2revb_hbm, out_hbm,
          idxa_all, idxr_all, ras, rrs, ris, ros, sas, srs, sis, sos):
        wid = lax.axis_index("s") * NC + lax.axis_index("c")
        wbase = wid * B_PER_W

        pltpu.sync_copy(b2a_hbm.at[pl.ds(wbase, B_PER_W)], idxa_all)
        pltpu.sync_copy(b2revb_hbm.at[pl.ds(wbase, B_PER_W)], idxr_all)

        def issue(g, b):
            iva = idxa_all.at[pl.ds(g * CH_B, CH_B)]
            ivr = idxr_all.at[pl.ds(g * CH_B, CH_B)]
            pltpu.async_copy(a_hbm.at[iva], ras[b], sas[b])
            pltpu.async_copy(mh_hbm.at[ivr], rrs[b], srs[b])
            pltpu.async_copy(inp_hbm.at[pl.ds(wbase + g * CH_B, CH_B)],
                             ris[b], sis[b])

        def drain_in(b):
            pltpu.make_async_copy(a_hbm.at[pl.ds(0, CH_B)], ras[b],
                                  sas[b]).wait()
            pltpu.make_async_copy(mh_hbm.at[pl.ds(0, CH_B)], rrs[b],
                                  srs[b]).wait()
            pltpu.make_async_copy(inp_hbm.at[pl.ds(0, CH_B)], ris[b],
                                  sis[b]).wait()

        def drain_out(b):
            pltpu.make_async_copy(ros[b], out_hbm.at[pl.ds(0, CH_B)],
                                  sos[b]).wait()

        for b in range(NB_B):
            issue(b, b)

        def step(g2, carry):
            for b in range(NB_B):
                g = NB_B * g2 + b
                drain_in(b)

                @pl.when(g >= NB_B)
                def _():
                    drain_out(b)

                def ew(r, c2):
                    for c in range(PKG):
                        s = pl.ds(c * 16, 16)
                        il, ih = _bf16_split(ris[b][r, s])
                        al, ah = _bf16_split(ras[b][r, s])
                        ml, mh_ = _bf16_split(rrs[b][r, s])
                        vl = jnp.maximum(il + al - ml, 0.0)
                        vh = jnp.maximum(ih + ah - mh_, 0.0)
                        ros[b][r, s] = _bf16_join_rta(vl, vh)
                    return c2

                lax.fori_loop(0, CH_B, ew, 0)
                pltpu.async_copy(ros[b],
                                 out_hbm.at[pl.ds(wbase + g * CH_B, CH_B)],
                                 sos[b])

                @pl.when(g + NB_B < N_CH_B)
                def _():
                    issue(g + NB_B, b)
            return carry

        lax.fori_loop(0, N_CH_B // NB_B, step, 0)
        for b in range(NB_B):
            drain_out(b)

    return k(inp, a_tab, mh_tab, b2a_p, b2revb_p)


def kernel(f_atoms, f_bonds, a2b, b2a, b2revb, mol_segment_ids,
           W_i, W_h, W_o, b_o):
    f_atoms_p = jnp.pad(f_atoms, ((0, A_PAD - N_ATOMS), (0, 0)))

    nbp = B_PAD - N_BONDS
    nap = A_PAD - N_ATOMS
    b2a_p = jnp.concatenate(
        [b2a.astype(jnp.int32),
         (jnp.arange(nbp, dtype=jnp.int32) * 131) % N_ATOMS])
    b2revb_p = jnp.concatenate(
        [b2revb.astype(jnp.int32),
         (jnp.arange(nbp, dtype=jnp.int32) * 97) % N_BONDS])
    a2b_p = jnp.concatenate(
        [a2b.astype(jnp.int32).reshape(-1),
         (jnp.arange(nap * MAX_NB, dtype=jnp.int32) * 193) % N_BONDS])
    ids_p = jnp.concatenate(
        [mol_segment_ids.astype(jnp.int32),
         jnp.full((nap,), M_PAD - 1, jnp.int32)]).reshape(1, A_PAD)

    inp = _mm(f_bonds, W_i, bm=3200, out_m=B_PAD)

    msg = inp
    relu_first = True
    for _ in range(DEPTH - 1):
        mh = _mm(msg, W_h, relu_in=relu_first, bm=4096)
        am = _gathersum_sc(msg, a2b_p, relu_in=relu_first)
        a_tab = _mm(am, W_h, bm=2048)
        msg = _update_sc(inp, a_tab, mh, b2a_p, b2revb_p)
        relu_first = False

    am_f = _gathersum_sc(msg, a2b_p, relu_in=False)
    mol = _readout(f_atoms_p, am_f, ids_p, W_o, b_o)
    return mol[:N_MOLS]


# speedup vs baseline: 1.1985x; 1.0665x over previous
import functools

import jax
import jax.numpy as jnp
from jax import lax
from jax.experimental import pallas as pl
from jax.experimental.pallas import tpu as pltpu
from jax.experimental.pallas import tpu_sc as plsc

N_ATOMS = 10000
N_BONDS = 160000
MAX_NB = 16
ATOM_FDIM = 256
BOND_FDIM = 272
HIDDEN = 256
DEPTH = 4
N_MOLS = 500

NC = 2
NS = 16
NW = NC * NS

A_PAD = 10240
B_PAD = 163840
M_PAD = 512

A_PER_W = A_PAD // NW
CH_A = 8
N_CH_A = A_PER_W // CH_A
NB_A = 4

B_PER_W = B_PAD // NW
CH_B = 40
N_CH_B = B_PER_W // CH_B
NB_B = 4

LG = HIDDEN // 16


def _rne_hi16(u):
    return (u + jnp.uint32(0x7FFF) + ((u >> 16) & jnp.uint32(1))) \
        & jnp.uint32(0xFFFF0000)


def _pack_tc(y):
    ua = jax.lax.bitcast_convert_type(y[:, :128], jnp.uint32)
    ub = jax.lax.bitcast_convert_type(y[:, 128:], jnp.uint32)
    return (_rne_hi16(ua) >> 16) | _rne_hi16(ub)


def _unpack_tc(u, relu):
    lo = jax.lax.bitcast_convert_type(u << 16, jnp.float32)
    hi = jax.lax.bitcast_convert_type(u & jnp.uint32(0xFFFF0000), jnp.float32)
    if relu:
        lo = jnp.maximum(lo, 0.0)
        hi = jnp.maximum(hi, 0.0)
    return lo.astype(jnp.bfloat16), hi.astype(jnp.bfloat16)


def _mm_body(x_ref, w_ref, o_ref, *, relu_in, packed_in):
    w = w_ref[...].astype(jnp.bfloat16)
    if packed_in:
        lo, hi = _unpack_tc(x_ref[...], relu_in)
        y = (jnp.dot(lo, w[:128], preferred_element_type=jnp.float32)
             + jnp.dot(hi, w[128:], preferred_element_type=jnp.float32))
    else:
        x = x_ref[...]
        if relu_in:
            x = jnp.maximum(x, 0.0)
        y = jnp.dot(x.astype(jnp.bfloat16), w,
                    preferred_element_type=jnp.float32)
    o_ref[...] = _pack_tc(y)


def _mm(x, w, relu_in=False, bm=512, out_m=None):
    m, k = x.shape
    n = w.shape[1]
    packed_in = x.dtype == jnp.uint32
    out_m = m if out_m is None else out_m
    return pl.pallas_call(
        functools.partial(_mm_body, relu_in=relu_in, packed_in=packed_in),
        grid=(m // bm,),
        in_specs=[pl.BlockSpec((bm, k), lambda i: (i, 0)),
                  pl.BlockSpec(w.shape, lambda i: (0, 0))],
        out_specs=pl.BlockSpec((bm, n // 2), lambda i: (i, 0)),
        out_shape=jax.ShapeDtypeStruct((out_m, n // 2), jnp.uint32),
    )(x, w)


def _readout_body(fa_ref, am_ref, ids_ref, wo_ref, bo_ref, o_ref,
                  sums_ref, cnts_ref):
    step = pl.program_id(0)

    @pl.when(step == 0)
    def _init():
        sums_ref[...] = jnp.zeros_like(sums_ref)
        cnts_ref[...] = jnp.zeros_like(cnts_ref)

    wo = wo_ref[...].astype(jnp.bfloat16)
    lo, hi = _unpack_tc(am_ref[...], False)
    hid = (jnp.dot(fa_ref[...].astype(jnp.bfloat16), wo[:ATOM_FDIM],
                   preferred_element_type=jnp.float32)
           + jnp.dot(lo, wo[ATOM_FDIM:ATOM_FDIM + 128],
                     preferred_element_type=jnp.float32)
           + jnp.dot(hi, wo[ATOM_FDIM + 128:],
                     preferred_element_type=jnp.float32)
           + bo_ref[...])
    hid = jnp.maximum(hid, 0.0)

    ids = ids_ref[...]
    mol_iota = lax.broadcasted_iota(jnp.int32, (M_PAD, ids.shape[1]), 0)
    onehot_t = (mol_iota == ids).astype(jnp.float32)
    sums_ref[...] += jnp.dot(onehot_t, hid, preferred_element_type=jnp.float32)
    cnts_ref[...] += jnp.broadcast_to(
        jnp.sum(onehot_t, axis=1, keepdims=True), cnts_ref.shape)

    @pl.when(step == pl.num_programs(0) - 1)
    def _fin():
        o_ref[...] = sums_ref[...] / jnp.maximum(cnts_ref[:, 0:1], 1.0)


def _readout(f_atoms_p, am_f, ids_p, w_o, b_o, bm=512):
    grid = (A_PAD // bm,)
    return pl.pallas_call(
        _readout_body,
        grid=grid,
        in_specs=[pl.BlockSpec((bm, ATOM_FDIM), lambda i: (i, 0)),
                  pl.BlockSpec((bm, HIDDEN // 2), lambda i: (i, 0)),
                  pl.BlockSpec((1, bm), lambda i: (0, i)),
                  pl.BlockSpec((ATOM_FDIM + HIDDEN, HIDDEN), lambda i: (0, 0)),
                  pl.BlockSpec((1, HIDDEN), lambda i: (0, 0))],
        out_specs=pl.BlockSpec((M_PAD, HIDDEN), lambda i: (0, 0)),
        out_shape=jax.ShapeDtypeStruct((M_PAD, HIDDEN), jnp.float32),
        scratch_shapes=[pltpu.VMEM((M_PAD, HIDDEN), jnp.float32),
                        pltpu.VMEM((M_PAD, 128), jnp.float32)],
    )(f_atoms_p, am_f, ids_p, w_o, b_o.reshape(1, HIDDEN))


PK = HIDDEN // 2
PKG = PK // 16


def _bf16_split(u):
    lo = plsc.bitcast(u << 16, jnp.float32)
    hi = plsc.bitcast(u & jnp.uint32(0xFFFF0000), jnp.float32)
    return lo, hi


def _bf16_join(lo, hi):
    ul = plsc.bitcast(lo, jnp.uint32)
    uh = plsc.bitcast(hi, jnp.uint32)
    return (_rne_hi16(ul) >> 16) | _rne_hi16(uh)


def _bf16_join_rta(lo, hi):
    ul = plsc.bitcast(lo, jnp.uint32) + jnp.uint32(0x8000)
    uh = plsc.bitcast(hi, jnp.uint32) + jnp.uint32(0x8000)
    return (ul >> 16) | (uh & jnp.uint32(0xFFFF0000))

def _gathersum_sc(tab, a2b_flat, relu_in):
    mesh = plsc.VectorSubcoreMesh(core_axis_name="c", subcore_axis_name="s")

    @functools.partial(
        pl.kernel, mesh=mesh,
        compiler_params=pltpu.CompilerParams(needs_layout_passes=False),
        out_type=jax.ShapeDtypeStruct((A_PAD, PK), jnp.uint32),
        scratch_types=[
            pltpu.VMEM((A_PER_W * MAX_NB,), jnp.int32),
            pltpu.VMEM((CH_A * MAX_NB, PK), jnp.uint32),
            pltpu.VMEM((CH_A * MAX_NB, PK), jnp.uint32),
            pltpu.VMEM((CH_A * MAX_NB, PK), jnp.uint32),
            pltpu.VMEM((CH_A * MAX_NB, PK), jnp.uint32),
            pltpu.VMEM((A_PER_W, PK), jnp.uint32),
            pltpu.SemaphoreType.DMA,
            pltpu.SemaphoreType.DMA,
            pltpu.SemaphoreType.DMA,
            pltpu.SemaphoreType.DMA,
        ],
    )
    def k(tab_hbm, idx_hbm, out_hbm, idx_all, rows0, rows1, rows2, rows3,
          out_all, gsem0, gsem1, gsem2, gsem3):
        wid = lax.axis_index("s") * NC + lax.axis_index("c")
        rows = (rows0, rows1, rows2, rows3)
        gsems = (gsem0, gsem1, gsem2, gsem3)

        pltpu.sync_copy(idx_hbm.at[pl.ds(wid * A_PER_W * MAX_NB,
                                         A_PER_W * MAX_NB)], idx_all)

        def issue(g, b):
            iv = idx_all.at[pl.ds(g * (CH_A * MAX_NB), CH_A * MAX_NB)]
            pltpu.async_copy(tab_hbm.at[iv], rows[b], gsems[b])

        def drain(b):
            pltpu.make_async_copy(tab_hbm.at[pl.ds(0, CH_A * MAX_NB)],
                                  rows[b], gsems[b]).wait()

        for b in range(NB_A):
            issue(b, b)

        def step(g2, carry):
            for b in range(NB_A):
                g = NB_A * g2 + b
                drain(b)

                def red(a, c2):
                    def ld(r, c):
                        lo, hi = _bf16_split(
                            rows[b][a * MAX_NB + r, pl.ds(c * 16, 16)])
                        if relu_in:
                            lo = jnp.maximum(lo, 0.0)
                            hi = jnp.maximum(hi, 0.0)
                        return lo, hi

                    for c0 in range(0, PKG, 4):
                        cg = range(c0, c0 + 4)
                        accs = {c: ld(0, c) for c in cg}
                        for r in range(1, MAX_NB):
                            for c in cg:
                                e, o = ld(r, c)
                                accs[c] = (accs[c][0] + e, accs[c][1] + o)
                        for c in cg:
                            out_all[g * CH_A + a, pl.ds(c * 16, 16)] = (
                                _bf16_join(accs[c][0], accs[c][1]))
                    return c2

                lax.fori_loop(0, CH_A, red, 0)

                @pl.when(g + NB_A < N_CH_A)
                def _():
                    issue(g + NB_A, b)
            return carry

        lax.fori_loop(0, N_CH_A // NB_A, step, 0)
        pltpu.sync_copy(out_all, out_hbm.at[pl.ds(wid * A_PER_W, A_PER_W)])

    return k(tab, a2b_flat)


_ST_BYTES = CH_B * (HIDDEN // 2) * 4


def _update_sc(inp, a_tab, mh_tab, b2a_p, b2revb_p):
    mesh = plsc.VectorSubcoreMesh(core_axis_name="c", subcore_axis_name="s")

    buf = lambda: [pltpu.VMEM((CH_B, PK), jnp.uint32) for _ in range(NB_B)]
    sem = lambda: [pltpu.SemaphoreType.DMA for _ in range(NB_B)]

    @functools.partial(
        pl.kernel, mesh=mesh,
        compiler_params=pltpu.CompilerParams(needs_layout_passes=False),
        out_type=jax.ShapeDtypeStruct((B_PAD, PK), jnp.uint32),
        scratch_types=[
            pltpu.VMEM((B_PER_W,), jnp.int32),
            pltpu.VMEM((B_PER_W,), jnp.int32),
            buf(), buf(), buf(), buf(),
            sem(), sem(), sem(), sem(),
        ],
    )
    def k(inp_hbm, a_hbm, mh_hbm, b2a_hbm, b2revb_hbm, out_hbm,
          idxa_all, idxr_all, ras, rrs, ris, ros, sas, srs, sis, sos):
        wid = lax.axis_index("s") * NC + lax.axis_index("c")
        wbase = wid * B_PER_W

        pltpu.sync_copy(b2a_hbm.at[pl.ds(wbase, B_PER_W)], idxa_all)
        pltpu.sync_copy(b2revb_hbm.at[pl.ds(wbase, B_PER_W)], idxr_all)

        def issue(g, b):
            iva = idxa_all.at[pl.ds(g * CH_B, CH_B)]
            ivr = idxr_all.at[pl.ds(g * CH_B, CH_B)]
            pltpu.async_copy(a_hbm.at[iva], ras[b], sas[b])
            pltpu.async_copy(mh_hbm.at[ivr], rrs[b], srs[b])
            pltpu.async_copy(inp_hbm.at[pl.ds(wbase + g * CH_B, CH_B)],
                             ris[b], sis[b])

        def drain_in(b):
            pltpu.make_async_copy(a_hbm.at[pl.ds(0, CH_B)], ras[b],
                                  sas[b]).wait()
            pltpu.make_async_copy(mh_hbm.at[pl.ds(0, CH_B)], rrs[b],
                                  srs[b]).wait()
            pltpu.make_async_copy(inp_hbm.at[pl.ds(0, CH_B)], ris[b],
                                  sis[b]).wait()

        def drain_out(b):
            pltpu.make_async_copy(ros[b], out_hbm.at[pl.ds(0, CH_B)],
                                  sos[b]).wait()

        for b in range(NB_B):
            issue(b, b)

        def step(g2, carry):
            for b in range(NB_B):
                g = NB_B * g2 + b
                drain_in(b)

                @pl.when(g >= NB_B)
                def _():
                    drain_out(b)

                def ew(r, c2):
                    for c in range(PKG):
                        s = pl.ds(c * 16, 16)
                        iv = plsc.bitcast(ris[b][r, s], jnp.bfloat16)
                        av = plsc.bitcast(ras[b][r, s], jnp.bfloat16)
                        mv = plsc.bitcast(rrs[b][r, s], jnp.bfloat16)
                        v = jnp.maximum(iv + av - mv, jnp.bfloat16(0.0))
                        ros[b][r, s] = plsc.bitcast(v, jnp.uint32)
                    return c2

                lax.fori_loop(0, CH_B, ew, 0)
                pltpu.async_copy(ros[b],
                                 out_hbm.at[pl.ds(wbase + g * CH_B, CH_B)],
                                 sos[b])

                @pl.when(g + NB_B < N_CH_B)
                def _():
                    issue(g + NB_B, b)
            return carry

        lax.fori_loop(0, N_CH_B // NB_B, step, 0)
        for b in range(NB_B):
            drain_out(b)

    return k(inp, a_tab, mh_tab, b2a_p, b2revb_p)


def kernel(f_atoms, f_bonds, a2b, b2a, b2revb, mol_segment_ids,
           W_i, W_h, W_o, b_o):
    f_atoms_p = jnp.pad(f_atoms, ((0, A_PAD - N_ATOMS), (0, 0)))

    nbp = B_PAD - N_BONDS
    nap = A_PAD - N_ATOMS
    b2a_p = jnp.concatenate(
        [b2a.astype(jnp.int32),
         (jnp.arange(nbp, dtype=jnp.int32) * 131) % N_ATOMS])
    b2revb_p = jnp.concatenate(
        [b2revb.astype(jnp.int32),
         (jnp.arange(nbp, dtype=jnp.int32) * 97) % N_BONDS])
    a2b_p = jnp.concatenate(
        [a2b.astype(jnp.int32).reshape(-1),
         (jnp.arange(nap * MAX_NB, dtype=jnp.int32) * 193) % N_BONDS])
    ids_p = jnp.concatenate(
        [mol_segment_ids.astype(jnp.int32),
         jnp.full((nap,), M_PAD - 1, jnp.int32)]).reshape(1, A_PAD)

    inp = _mm(f_bonds, W_i, bm=3200, out_m=B_PAD)

    msg = inp
    relu_first = True
    for _ in range(DEPTH - 1):
        mh = _mm(msg, W_h, relu_in=relu_first, bm=4096)
        am = _gathersum_sc(msg, a2b_p, relu_in=relu_first)
        a_tab = _mm(am, W_h, bm=2048)
        msg = _update_sc(inp, a_tab, mh, b2a_p, b2revb_p)
        relu_first = False

    am_f = _gathersum_sc(msg, a2b_p, relu_in=False)
    mol = _readout(f_atoms_p, am_f, ids_p, W_o, b_o)
    return mol[:N_MOLS]


# speedup vs baseline: 1.2264x; 1.0233x over previous
import functools

import jax
import jax.numpy as jnp
from jax import lax
from jax.experimental import pallas as pl
from jax.experimental.pallas import tpu as pltpu
from jax.experimental.pallas import tpu_sc as plsc

N_ATOMS = 10000
N_BONDS = 160000
MAX_NB = 16
ATOM_FDIM = 256
BOND_FDIM = 272
HIDDEN = 256
DEPTH = 4
N_MOLS = 500

NC = 2
NS = 16
NW = NC * NS

A_PAD = 10240
B_PAD = 163840
M_PAD = 512

A_PER_W = A_PAD // NW
CH_A = 8
N_CH_A = A_PER_W // CH_A
NB_A = 4

B_PER_W = B_PAD // NW
CH_B = 40
N_CH_B = B_PER_W // CH_B
NB_B = 4

LG = HIDDEN // 16


def _rne_hi16(u):
    return (u + jnp.uint32(0x7FFF) + ((u >> 16) & jnp.uint32(1))) \
        & jnp.uint32(0xFFFF0000)


def _pack_tc(y):
    ua = jax.lax.bitcast_convert_type(y[:, :128], jnp.uint32)
    ub = jax.lax.bitcast_convert_type(y[:, 128:], jnp.uint32)
    return (_rne_hi16(ua) >> 16) | _rne_hi16(ub)


def _unpack_tc(u, relu):
    lo = jax.lax.bitcast_convert_type(u << 16, jnp.float32)
    hi = jax.lax.bitcast_convert_type(u & jnp.uint32(0xFFFF0000), jnp.float32)
    if relu:
        lo = jnp.maximum(lo, 0.0)
        hi = jnp.maximum(hi, 0.0)
    return lo.astype(jnp.bfloat16), hi.astype(jnp.bfloat16)


def _mm_body(x_ref, w_ref, o_ref, *, relu_in, packed_in):
    w = w_ref[...].astype(jnp.bfloat16)
    if packed_in:
        lo, hi = _unpack_tc(x_ref[...], relu_in)
        y = (jnp.dot(lo, w[:128], preferred_element_type=jnp.float32)
             + jnp.dot(hi, w[128:], preferred_element_type=jnp.float32))
    else:
        x = x_ref[...]
        if relu_in:
            x = jnp.maximum(x, 0.0)
        y = jnp.dot(x.astype(jnp.bfloat16), w,
                    preferred_element_type=jnp.float32)
    o_ref[...] = _pack_tc(y)


def _mm(x, w, relu_in=False, bm=512, out_m=None):
    m, k = x.shape
    n = w.shape[1]
    packed_in = x.dtype == jnp.uint32
    out_m = m if out_m is None else out_m
    return pl.pallas_call(
        functools.partial(_mm_body, relu_in=relu_in, packed_in=packed_in),
        grid=(m // bm,),
        in_specs=[pl.BlockSpec((bm, k), lambda i: (i, 0)),
                  pl.BlockSpec(w.shape, lambda i: (0, 0))],
        out_specs=pl.BlockSpec((bm, n // 2), lambda i: (i, 0)),
        out_shape=jax.ShapeDtypeStruct((out_m, n // 2), jnp.uint32),
    )(x, w)


def _readout_body(fa_ref, am_ref, ids_ref, wo_ref, bo_ref, o_ref,
                  sums_ref, cnts_ref):
    step = pl.program_id(0)

    @pl.when(step == 0)
    def _init():
        sums_ref[...] = jnp.zeros_like(sums_ref)
        cnts_ref[...] = jnp.zeros_like(cnts_ref)

    wo = wo_ref[...].astype(jnp.bfloat16)
    lo, hi = _unpack_tc(am_ref[...], False)
    hid = (jnp.dot(fa_ref[...].astype(jnp.bfloat16), wo[:ATOM_FDIM],
                   preferred_element_type=jnp.float32)
           + jnp.dot(lo, wo[ATOM_FDIM:ATOM_FDIM + 128],
                     preferred_element_type=jnp.float32)
           + jnp.dot(hi, wo[ATOM_FDIM + 128:],
                     preferred_element_type=jnp.float32)
           + bo_ref[...])
    hid = jnp.maximum(hid, 0.0)

    ids = ids_ref[...]
    mol_iota = lax.broadcasted_iota(jnp.int32, (M_PAD, ids.shape[1]), 0)
    onehot_t = (mol_iota == ids).astype(jnp.float32)
    sums_ref[...] += jnp.dot(onehot_t, hid, preferred_element_type=jnp.float32)
    cnts_ref[...] += jnp.broadcast_to(
        jnp.sum(onehot_t, axis=1, keepdims=True), cnts_ref.shape)

    @pl.when(step == pl.num_programs(0) - 1)
    def _fin():
        o_ref[...] = sums_ref[...] / jnp.maximum(cnts_ref[:, 0:1], 1.0)


def _readout(f_atoms_p, am_f, ids_p, w_o, b_o, bm=512):
    grid = (A_PAD // bm,)
    return pl.pallas_call(
        _readout_body,
        grid=grid,
        in_specs=[pl.BlockSpec((bm, ATOM_FDIM), lambda i: (i, 0)),
                  pl.BlockSpec((bm, HIDDEN // 2), lambda i: (i, 0)),
                  pl.BlockSpec((1, bm), lambda i: (0, i)),
                  pl.BlockSpec((ATOM_FDIM + HIDDEN, HIDDEN), lambda i: (0, 0)),
                  pl.BlockSpec((1, HIDDEN), lambda i: (0, 0))],
        out_specs=pl.BlockSpec((M_PAD, HIDDEN), lambda i: (0, 0)),
        out_shape=jax.ShapeDtypeStruct((M_PAD, HIDDEN), jnp.float32),
        scratch_shapes=[pltpu.VMEM((M_PAD, HIDDEN), jnp.float32),
                        pltpu.VMEM((M_PAD, 128), jnp.float32)],
    )(f_atoms_p, am_f, ids_p, w_o, b_o.reshape(1, HIDDEN))


PK = HIDDEN // 2
PKG = PK // 16


def _bf16_split(u):
    lo = plsc.bitcast(u << 16, jnp.float32)
    hi = plsc.bitcast(u & jnp.uint32(0xFFFF0000), jnp.float32)
    return lo, hi


def _bf16_join(lo, hi):
    ul = plsc.bitcast(lo, jnp.uint32)
    uh = plsc.bitcast(hi, jnp.uint32)
    return (_rne_hi16(ul) >> 16) | _rne_hi16(uh)


def _bf16_join_rta(lo, hi):
    ul = plsc.bitcast(lo, jnp.uint32) + jnp.uint32(0x8000)
    uh = plsc.bitcast(hi, jnp.uint32) + jnp.uint32(0x8000)
    return (ul >> 16) | (uh & jnp.uint32(0xFFFF0000))

def _gathersum_sc(tab, a2b_flat, relu_in):
    mesh = plsc.VectorSubcoreMesh(core_axis_name="c", subcore_axis_name="s")

    @functools.partial(
        pl.kernel, mesh=mesh,
        compiler_params=pltpu.CompilerParams(needs_layout_passes=False),
        out_type=jax.ShapeDtypeStruct((A_PAD, PK), jnp.uint32),
        scratch_types=[
            pltpu.VMEM((A_PER_W * MAX_NB,), jnp.int32),
            pltpu.VMEM((CH_A * MAX_NB, PK), jnp.uint32),
            pltpu.VMEM((CH_A * MAX_NB, PK), jnp.uint32),
            pltpu.VMEM((CH_A * MAX_NB, PK), jnp.uint32),
            pltpu.VMEM((CH_A * MAX_NB, PK), jnp.uint32),
            pltpu.VMEM((A_PER_W, PK), jnp.uint32),
            pltpu.SemaphoreType.DMA,
            pltpu.SemaphoreType.DMA,
            pltpu.SemaphoreType.DMA,
            pltpu.SemaphoreType.DMA,
        ],
    )
    def k(tab_hbm, idx_hbm, out_hbm, idx_all, rows0, rows1, rows2, rows3,
          out_all, gsem0, gsem1, gsem2, gsem3):
        wid = lax.axis_index("s") * NC + lax.axis_index("c")
        rows = (rows0, rows1, rows2, rows3)
        gsems = (gsem0, gsem1, gsem2, gsem3)

        pltpu.sync_copy(idx_hbm.at[pl.ds(wid * A_PER_W * MAX_NB,
                                         A_PER_W * MAX_NB)], idx_all)

        def issue(g, b):
            iv = idx_all.at[pl.ds(g * (CH_A * MAX_NB), CH_A * MAX_NB)]
            pltpu.async_copy(tab_hbm.at[iv], rows[b], gsems[b])

        def drain(b):
            pltpu.make_async_copy(tab_hbm.at[pl.ds(0, CH_A * MAX_NB)],
                                  rows[b], gsems[b]).wait()

        for b in range(NB_A):
            issue(b, b)

        def step(g2, carry):
            for b in range(NB_A):
                g = NB_A * g2 + b
                drain(b)

                def red(a, c2):
                    def ld(r, c):
                        lo, hi = _bf16_split(
                            rows[b][a * MAX_NB + r, pl.ds(c * 16, 16)])
                        if relu_in:
                            lo = jnp.maximum(lo, 0.0)
                            hi = jnp.maximum(hi, 0.0)
                        return lo, hi

                    for c0 in range(0, PKG, 4):
                        cg = range(c0, c0 + 4)
                        accs = {c: ld(0, c) for c in cg}
                        for r in range(1, MAX_NB):
                            for c in cg:
                                e, o = ld(r, c)
                                accs[c] = (accs[c][0] + e, accs[c][1] + o)
                        for c in cg:
                            out_all[g * CH_A + a, pl.ds(c * 16, 16)] = (
                                _bf16_join(accs[c][0], accs[c][1]))
                    return c2

                lax.fori_loop(0, CH_A, red, 0)

                @pl.when(g + NB_A < N_CH_A)
                def _():
                    issue(g + NB_A, b)
            return carry

        lax.fori_loop(0, N_CH_A // NB_A, step, 0)
        pltpu.sync_copy(out_all, out_hbm.at[pl.ds(wid * A_PER_W, A_PER_W)])

    return k(tab, a2b_flat)


_ST_BYTES = CH_B * (HIDDEN // 2) * 4


def _update_sc(inp, a_tab, mh_tab, b2a_p, b2revb_p):
    mesh = plsc.VectorSubcoreMesh(core_axis_name="c", subcore_axis_name="s")

    buf = lambda: [pltpu.VMEM((CH_B, PK), jnp.uint32) for _ in range(NB_B)]
    sem = lambda: [pltpu.SemaphoreType.DMA for _ in range(NB_B)]

    @functools.partial(
        pl.kernel, mesh=mesh,
        compiler_params=pltpu.CompilerParams(needs_layout_passes=False),
        out_type=jax.ShapeDtypeStruct((B_PAD, PK), jnp.uint32),
        scratch_types=[
            pltpu.VMEM((B_PER_W,), jnp.int32),
            pltpu.VMEM((B_PER_W,), jnp.int32),
            buf(), buf(), buf(), buf(),
            sem(), sem(), sem(), sem(),
        ],
    )
    def k(inp_hbm, a_hbm, mh_hbm, b2a_hbm, b2revb_hbm, out_hbm,
          idxa_all, idxr_all, ras, rrs, ris, ros, sas, srs, sis, sos):
        wid = lax.axis_index("s") * NC + lax.axis_index("c")
        wbase = wid * B_PER_W

        pltpu.sync_copy(b2a_hbm.at[pl.ds(wbase, B_PER_W)], idxa_all)
        pltpu.sync_copy(b2revb_hbm.at[pl.ds(wbase, B_PER_W)], idxr_all)

        def issue(g, b):
            iva = idxa_all.at[pl.ds(g * CH_B, CH_B)]
            ivr = idxr_all.at[pl.ds(g * CH_B, CH_B)]
            pltpu.async_copy(a_hbm.at[iva], ras[b], sas[b])
            pltpu.async_copy(mh_hbm.at[ivr], rrs[b], srs[b])
            pltpu.async_copy(inp_hbm.at[pl.ds(wbase + g * CH_B, CH_B)],
                             ris[b], sis[b])

        def drain_in(b):
            pltpu.make_async_copy(a_hbm.at[pl.ds(0, CH_B)], ras[b],
                                  sas[b]).wait()
            pltpu.make_async_copy(mh_hbm.at[pl.ds(0, CH_B)], rrs[b],
                                  srs[b]).wait()
            pltpu.make_async_copy(inp_hbm.at[pl.ds(0, CH_B)], ris[b],
                                  sis[b]).wait()

        def drain_out(b):
            pltpu.make_async_copy(ros[b], out_hbm.at[pl.ds(0, CH_B)],
                                  sos[b]).wait()

        for b in range(NB_B):
            issue(b, b)

        def step(g2, carry):
            for b in range(NB_B):
                g = NB_B * g2 + b
                drain_in(b)

                @pl.when(g >= NB_B)
                def _():
                    drain_out(b)

                def ew(r, c2):
                    for c in range(PKG):
                        s = pl.ds(c * 16, 16)
                        iv = plsc.bitcast(ris[b][r, s], jnp.bfloat16)
                        av = plsc.bitcast(ras[b][r, s], jnp.bfloat16)
                        mv = plsc.bitcast(rrs[b][r, s], jnp.bfloat16)
                        v = jnp.maximum(iv + av - mv, jnp.bfloat16(0.0))
                        ros[b][r, s] = plsc.bitcast(v, jnp.uint32)
                    return c2

                lax.fori_loop(0, CH_B, ew, 0)
                pltpu.async_copy(ros[b],
                                 out_hbm.at[pl.ds(wbase + g * CH_B, CH_B)],
                                 sos[b])

                @pl.when(g + NB_B < N_CH_B)
                def _():
                    issue(g + NB_B, b)
            return carry

        lax.fori_loop(0, N_CH_B // NB_B, step, 0)
        for b in range(NB_B):
            drain_out(b)

    return k(inp, a_tab, mh_tab, b2a_p, b2revb_p)


def kernel(f_atoms, f_bonds, a2b, b2a, b2revb, mol_segment_ids,
           W_i, W_h, W_o, b_o):
    f_atoms_p = jnp.pad(f_atoms, ((0, A_PAD - N_ATOMS), (0, 0)))

    nbp = B_PAD - N_BONDS
    nap = A_PAD - N_ATOMS
    b2a_p = jnp.concatenate(
        [b2a.astype(jnp.int32),
         (jnp.arange(nbp, dtype=jnp.int32) * 131) % N_ATOMS])
    b2revb_p = jnp.concatenate(
        [b2revb.astype(jnp.int32),
         (jnp.arange(nbp, dtype=jnp.int32) * 97) % N_BONDS])
    a2b_p = jnp.concatenate(
        [a2b.astype(jnp.int32).reshape(-1),
         (jnp.arange(nap * MAX_NB, dtype=jnp.int32) * 193) % N_BONDS])
    ids_p = jnp.concatenate(
        [mol_segment_ids.astype(jnp.int32),
         jnp.full((nap,), M_PAD - 1, jnp.int32)]).reshape(1, A_PAD)

    inp = _mm(f_bonds, W_i, bm=6400, out_m=B_PAD)

    msg = inp
    relu_first = True
    for _ in range(DEPTH - 1):
        mh = _mm(msg, W_h, relu_in=relu_first, bm=8192)
        am = _gathersum_sc(msg, a2b_p, relu_in=relu_first)
        a_tab = _mm(am, W_h, bm=2048)
        msg = _update_sc(inp, a_tab, mh, b2a_p, b2revb_p)
        relu_first = False

    am_f = _gathersum_sc(msg, a2b_p, relu_in=False)
    mol = _readout(f_atoms_p, am_f, ids_p, W_o, b_o)
    return mol[:N_MOLS]
